# baseline (speedup 1.0000x reference)
import functools

import jax
import jax.numpy as jnp
from jax import lax
from jax.experimental import pallas as pl
from jax.experimental.pallas import tpu as pltpu
from jax.experimental.pallas import tpu_sc as plsc

N_NODES = 10000
N_EDGES = 320000
D_FEAT = 128
D_EDGE = 16
N_GRAPHS = 8
LATENT = 128

_BN = 1000


def _tables_body(x_ref, bt_ref, u_ref, wa_ref, wb_ref, wu_ref, b1_ref,
                 a_ref, b_ref):
    x = x_ref[...]
    uw = jnp.dot(u_ref[...], wu_ref[...], preferred_element_type=jnp.float32)
    oh = (bt_ref[...] == lax.broadcasted_iota(jnp.int32, (_BN, N_GRAPHS), 1)
          ).astype(jnp.float32)
    a_ref[...] = (jnp.dot(x, wa_ref[...], preferred_element_type=jnp.float32)
                  + jnp.dot(oh, uw, preferred_element_type=jnp.float32)
                  + b1_ref[...])
    b_ref[...] = jnp.dot(x, wb_ref[...], preferred_element_type=jnp.float32)


def _node_tables(x, batch2d, u, wa, wb, wu, b1):
    grid = (N_NODES // _BN,)
    return pl.pallas_call(
        _tables_body,
        grid=grid,
        in_specs=[
            pl.BlockSpec((_BN, D_FEAT), lambda i: (i, 0)),
            pl.BlockSpec((_BN, 1), lambda i: (i, 0)),
            pl.BlockSpec((N_GRAPHS, D_FEAT), lambda i: (0, 0)),
            pl.BlockSpec((D_FEAT, LATENT), lambda i: (0, 0)),
            pl.BlockSpec((D_FEAT, LATENT), lambda i: (0, 0)),
            pl.BlockSpec((D_FEAT, LATENT), lambda i: (0, 0)),
            pl.BlockSpec((1, LATENT), lambda i: (0, 0)),
        ],
        out_specs=[
            pl.BlockSpec((_BN, LATENT), lambda i: (i, 0)),
            pl.BlockSpec((_BN, LATENT), lambda i: (i, 0)),
        ],
        out_shape=[
            jax.ShapeDtypeStruct((N_NODES, LATENT), jnp.float32),
            jax.ShapeDtypeStruct((N_NODES, LATENT), jnp.float32),
        ],
    )(x, batch2d, u, wa, wb, wu, b1)


_NC = 2
_NS = 16
_NW = _NC * _NS
_CB = 128
_K = 4
_E_SLICE = N_EDGES // _K
_NCHUNK = 20
_PER_W = _NCHUNK * _CB
_ES_PAD = _NW * _PER_W


@functools.cache
def _build_gather():
    mesh = plsc.VectorSubcoreMesh(core_axis_name="c", subcore_axis_name="s",
                                  num_cores=_NC, num_subcores=_NS)

    @functools.partial(
        pl.kernel,
        mesh=mesh,
        out_type=jax.ShapeDtypeStruct((_ES_PAD, LATENT), jnp.int32),
        scratch_types=[
            pltpu.VMEM((_NCHUNK, _CB), jnp.int32),
            pltpu.VMEM((_NCHUNK, _CB), jnp.int32),
            pltpu.VMEM((_CB, LATENT), jnp.float32),
            pltpu.VMEM((_CB, LATENT), jnp.float32),
            pltpu.VMEM((_CB, LATENT), jnp.float32),
            pltpu.VMEM((_CB, LATENT), jnp.float32),
            pltpu.VMEM((_CB, LATENT), jnp.int32),
            pltpu.VMEM((_CB, LATENT), jnp.int32),
            pltpu.SemaphoreType.DMA,
            pltpu.SemaphoreType.DMA,
            pltpu.SemaphoreType.DMA,
            pltpu.SemaphoreType.DMA,
        ],
        compiler_params=pltpu.CompilerParams(needs_layout_passes=False),
    )
    def _gather_tables(a_hbm, b_hbm, src_hbm, dst_hbm, gp_hbm,
                       si_v, di_v, bufa0, bufb0, bufa1, bufb1,
                       bufp0, bufp1, s0, s1, w0, w1):
        wid = lax.axis_index("s") * _NC + lax.axis_index("c")
        base = wid * _PER_W
        pltpu.sync_copy(src_hbm.at[wid], si_v)
        pltpu.sync_copy(dst_hbm.at[wid], di_v)
        half = LATENT // 2

        def gathers(c, bufa, bufb, sem):
            pltpu.async_copy(a_hbm.at[si_v.at[c]], bufa, sem)
            pltpu.async_copy(b_hbm.at[di_v.at[c]], bufb, sem)

        def drain_g(c, bufa, bufb, sem):
            pltpu.make_async_copy(a_hbm.at[si_v.at[c]], bufa, sem).wait()
            pltpu.make_async_copy(b_hbm.at[di_v.at[c]], bufb, sem).wait()

        def pack_rows(bufa, bufb, bufp):
            rnd = jnp.uint32(0x8000)
            himask = jnp.uint32(0xFFFF0000)

            @plsc.parallel_loop(0, _CB, unroll=8)
            def _(r):
                for off, srcbuf in ((0, bufa), (half, bufb)):
                    for q in range(4):
                        a = plsc.bitcast(srcbuf[r, pl.ds(16 * q, 16)],
                                         jnp.uint32)
                        b = plsc.bitcast(
                            srcbuf[r, pl.ds(half + 16 * q, 16)], jnp.uint32)
                        lo = (a + rnd) >> 16
                        hi = (b + rnd) & himask
                        bufp[r, pl.ds(off + 16 * q, 16)] = plsc.bitcast(
                            lo | hi, jnp.int32)

        def writeback(c, bufp, sem):
            pltpu.async_copy(bufp, gp_hbm.at[pl.ds(base + c * _CB, _CB)], sem)

        def drain_w(c, bufp, sem):
            pltpu.make_async_copy(
                bufp, gp_hbm.at[pl.ds(base + c * _CB, _CB)], sem).wait()

        gathers(0, bufa0, bufb0, s0)
        gathers(1, bufa1, bufb1, s1)

        def body(i, carry):
            c0 = 2 * i
            c1 = c0 + 1
            drain_g(c0, bufa0, bufb0, s0)

            @pl.when(i > 0)
            def _():
                drain_w(c0 - 2, bufp0, w0)

            pack_rows(bufa0, bufb0, bufp0)
            writeback(c0, bufp0, w0)

            @pl.when(i < _NCHUNK // 2 - 1)
            def _():
                gathers(c0 + 2, bufa0, bufb0, s0)

            drain_g(c1, bufa1, bufb1, s1)

            @pl.when(i > 0)
            def _():
                drain_w(c1 - 2, bufp1, w1)

            pack_rows(bufa1, bufb1, bufp1)
            writeback(c1, bufp1, w1)

            @pl.when(i < _NCHUNK // 2 - 1)
            def _():
                gathers(c1 + 2, bufa1, bufb1, s1)

            return carry

        lax.fori_loop(0, _NCHUNK // 2, body, 0)
        drain_w(_NCHUNK - 2, bufp0, w0)
        drain_w(_NCHUNK - 1, bufp1, w1)

    return _gather_tables


_BE = 3200
_BLK_SLICE = _E_SLICE // _BE


def _unpack_bf16x2(p):
    pu = lax.bitcast_convert_type(p, jnp.uint32)
    lo = lax.bitcast_convert_type(pu << 16, jnp.float32)
    hi = lax.bitcast_convert_type(pu & jnp.uint32(0xFFFF0000), jnp.float32)
    return jnp.concatenate([lo, hi], axis=1)


def _mlp_body_first(gp_ref, ea_ref, we_ref, w2_ref, b2_ref, o_ref):
    gp = gp_ref[...]
    h = (_unpack_bf16x2(gp[:, :LATENT // 2])
         + _unpack_bf16x2(gp[:, LATENT // 2:])
         + lax.dot_general(ea_ref[...], we_ref[...],
                           ((((0,), (0,)), ((), ()))),
                           preferred_element_type=jnp.float32))
    h = jnp.maximum(h, 0.0)
    o_ref[...] = jnp.maximum(
        jnp.dot(h, w2_ref[...], preferred_element_type=jnp.float32)
        + b2_ref[...], 0.0)


def _mlp_body_chained(prev_ref, gp_ref, ea_ref, we_ref, w2_ref,
                      b2_ref, o_ref):
    del prev_ref
    _mlp_body_first(gp_ref, ea_ref, we_ref, w2_ref, b2_ref, o_ref)


@functools.cache
def _build_mlp(k):
    row0 = k * _BLK_SLICE
    gather_specs = [
        pl.BlockSpec((_BE, LATENT), lambda i: (i, 0)),
    ]
    shared_specs = [
        pl.BlockSpec((D_EDGE, _BE), lambda i, row0=row0: (0, row0 + i)),
        pl.BlockSpec((D_EDGE, LATENT), lambda i: (0, 0)),
        pl.BlockSpec((LATENT, LATENT), lambda i: (0, 0)),
        pl.BlockSpec((1, LATENT), lambda i: (0, 0)),
    ]
    out_spec = pl.BlockSpec((_BE, LATENT), lambda i, row0=row0: (row0 + i, 0))
    out_shape = jax.ShapeDtypeStruct((N_EDGES, LATENT), jnp.float32)
    if k == 0:
        return pl.pallas_call(
            _mlp_body_first,
            grid=(_BLK_SLICE,),
            in_specs=gather_specs + shared_specs,
            out_specs=out_spec,
            out_shape=out_shape,
        )
    return pl.pallas_call(
        _mlp_body_chained,
        grid=(_BLK_SLICE,),
        in_specs=[pl.BlockSpec(memory_space=pl.ANY)] + gather_specs
        + shared_specs,
        out_specs=out_spec,
        out_shape=out_shape,
        input_output_aliases={0: 0},
    )


def kernel(x, edge_index, edge_attr, u, batch, W1, b1, W2, b2):
    src_all = edge_index[0].astype(jnp.int32)
    dst_all = edge_index[1].astype(jnp.int32)
    pad = (jnp.arange(_ES_PAD - _E_SLICE, dtype=jnp.int32) % N_NODES)
    batch2d = batch.astype(jnp.int32).reshape(N_NODES, 1)
    wa = W1[0:D_FEAT]
    wb = W1[D_FEAT:2 * D_FEAT]
    we = W1[2 * D_FEAT:2 * D_FEAT + D_EDGE]
    wu = W1[2 * D_FEAT + D_EDGE:]
    a_tab, b_tab = _node_tables(x, batch2d, u, wa, wb, wu,
                                b1.reshape(1, LATENT))
    gather = _build_gather()
    b2r = b2.reshape(1, LATENT)
    ea_t = edge_attr.T
    out = None
    for k in range(_K):
        lo = k * _E_SLICE
        src = jnp.concatenate([src_all[lo:lo + _E_SLICE], pad]
                              ).reshape(_NW, _NCHUNK, _CB)
        dst = jnp.concatenate([dst_all[lo:lo + _E_SLICE], pad]
                              ).reshape(_NW, _NCHUNK, _CB)
        gp = gather(a_tab, b_tab, src, dst)
        if k == 0:
            out = _build_mlp(0)(gp, ea_t, we, W2, b2r)
        else:
            out = _build_mlp(k)(out, gp, ea_t, we, W2, b2r)
    return out


# speedup vs baseline: 1.0139x; 1.0139x over previous
import functools

import jax
import jax.numpy as jnp
from jax import lax
from jax.experimental import pallas as pl
from jax.experimental.pallas import tpu as pltpu
from jax.experimental.pallas import tpu_sc as plsc

N_NODES = 10000
N_EDGES = 320000
D_FEAT = 128
D_EDGE = 16
N_GRAPHS = 8
LATENT = 128

_BN = 1000


def _tables_body(x_ref, bt_ref, u_ref, wa_ref, wb_ref, wu_ref, b1_ref,
                 a_ref, b_ref):
    x = x_ref[...]
    uw = jnp.dot(u_ref[...], wu_ref[...], preferred_element_type=jnp.float32)
    oh = (bt_ref[...] == lax.broadcasted_iota(jnp.int32, (_BN, N_GRAPHS), 1)
          ).astype(jnp.float32)
    a_ref[...] = (jnp.dot(x, wa_ref[...], preferred_element_type=jnp.float32)
                  + jnp.dot(oh, uw, preferred_element_type=jnp.float32)
                  + b1_ref[...])
    b_ref[...] = jnp.dot(x, wb_ref[...], preferred_element_type=jnp.float32)


def _node_tables(x, batch2d, u, wa, wb, wu, b1):
    grid = (N_NODES // _BN,)
    return pl.pallas_call(
        _tables_body,
        grid=grid,
        in_specs=[
            pl.BlockSpec((_BN, D_FEAT), lambda i: (i, 0)),
            pl.BlockSpec((_BN, 1), lambda i: (i, 0)),
            pl.BlockSpec((N_GRAPHS, D_FEAT), lambda i: (0, 0)),
            pl.BlockSpec((D_FEAT, LATENT), lambda i: (0, 0)),
            pl.BlockSpec((D_FEAT, LATENT), lambda i: (0, 0)),
            pl.BlockSpec((D_FEAT, LATENT), lambda i: (0, 0)),
            pl.BlockSpec((1, LATENT), lambda i: (0, 0)),
        ],
        out_specs=[
            pl.BlockSpec((_BN, LATENT), lambda i: (i, 0)),
            pl.BlockSpec((_BN, LATENT), lambda i: (i, 0)),
        ],
        out_shape=[
            jax.ShapeDtypeStruct((N_NODES, LATENT), jnp.float32),
            jax.ShapeDtypeStruct((N_NODES, LATENT), jnp.float32),
        ],
    )(x, batch2d, u, wa, wb, wu, b1)


_NC = 2
_NS = 16
_NW = _NC * _NS
_CB = 128
_K = 4
_SLICE_REAL = (96000, 96000, 96000, 32000)
_SLICE_NCHUNK = (24, 24, 24, 8)


@functools.cache
def _build_gather(nchunk):
    per_w = nchunk * _CB
    mesh = plsc.VectorSubcoreMesh(core_axis_name="c", subcore_axis_name="s",
                                  num_cores=_NC, num_subcores=_NS)

    @functools.partial(
        pl.kernel,
        mesh=mesh,
        out_type=jax.ShapeDtypeStruct((_NW * per_w, LATENT), jnp.int32),
        scratch_types=[
            pltpu.VMEM((nchunk, _CB), jnp.int32),
            pltpu.VMEM((nchunk, _CB), jnp.int32),
            pltpu.VMEM((_CB, LATENT), jnp.float32),
            pltpu.VMEM((_CB, LATENT), jnp.float32),
            pltpu.VMEM((_CB, LATENT), jnp.float32),
            pltpu.VMEM((_CB, LATENT), jnp.float32),
            pltpu.VMEM((_CB, LATENT), jnp.int32),
            pltpu.VMEM((_CB, LATENT), jnp.int32),
            pltpu.SemaphoreType.DMA,
            pltpu.SemaphoreType.DMA,
            pltpu.SemaphoreType.DMA,
            pltpu.SemaphoreType.DMA,
        ],
        compiler_params=pltpu.CompilerParams(needs_layout_passes=False),
    )
    def _gather_tables(a_hbm, b_hbm, src_hbm, dst_hbm, gp_hbm,
                       si_v, di_v, bufa0, bufb0, bufa1, bufb1,
                       bufp0, bufp1, s0, s1, w0, w1):
        wid = lax.axis_index("s") * _NC + lax.axis_index("c")
        base = wid * per_w
        pltpu.sync_copy(src_hbm.at[wid], si_v)
        pltpu.sync_copy(dst_hbm.at[wid], di_v)
        half = LATENT // 2

        def gathers(c, bufa, bufb, sem):
            pltpu.async_copy(a_hbm.at[si_v.at[c]], bufa, sem)
            pltpu.async_copy(b_hbm.at[di_v.at[c]], bufb, sem)

        def drain_g(c, bufa, bufb, sem):
            pltpu.make_async_copy(a_hbm.at[si_v.at[c]], bufa, sem).wait()
            pltpu.make_async_copy(b_hbm.at[di_v.at[c]], bufb, sem).wait()

        def pack_rows(bufa, bufb, bufp):
            rnd = jnp.uint32(0x8000)
            himask = jnp.uint32(0xFFFF0000)

            @plsc.parallel_loop(0, _CB, unroll=8)
            def _(r):
                for off, srcbuf in ((0, bufa), (half, bufb)):
                    for q in range(4):
                        a = plsc.bitcast(srcbuf[r, pl.ds(16 * q, 16)],
                                         jnp.uint32)
                        b = plsc.bitcast(
                            srcbuf[r, pl.ds(half + 16 * q, 16)], jnp.uint32)
                        lo = (a + rnd) >> 16
                        hi = (b + rnd) & himask
                        bufp[r, pl.ds(off + 16 * q, 16)] = plsc.bitcast(
                            lo | hi, jnp.int32)

        def writeback(c, bufp, sem):
            pltpu.async_copy(bufp, gp_hbm.at[pl.ds(base + c * _CB, _CB)], sem)

        def drain_w(c, bufp, sem):
            pltpu.make_async_copy(
                bufp, gp_hbm.at[pl.ds(base + c * _CB, _CB)], sem).wait()

        gathers(0, bufa0, bufb0, s0)
        gathers(1, bufa1, bufb1, s1)

        def body(i, carry):
            c0 = 2 * i
            c1 = c0 + 1
            drain_g(c0, bufa0, bufb0, s0)

            @pl.when(i > 0)
            def _():
                drain_w(c0 - 2, bufp0, w0)

            pack_rows(bufa0, bufb0, bufp0)
            writeback(c0, bufp0, w0)

            @pl.when(i < nchunk // 2 - 1)
            def _():
                gathers(c0 + 2, bufa0, bufb0, s0)

            drain_g(c1, bufa1, bufb1, s1)

            @pl.when(i > 0)
            def _():
                drain_w(c1 - 2, bufp1, w1)

            pack_rows(bufa1, bufb1, bufp1)
            writeback(c1, bufp1, w1)

            @pl.when(i < nchunk // 2 - 1)
            def _():
                gathers(c1 + 2, bufa1, bufb1, s1)

            return carry

        lax.fori_loop(0, nchunk // 2, body, 0)
        drain_w(nchunk - 2, bufp0, w0)
        drain_w(nchunk - 1, bufp1, w1)

    return _gather_tables


_BE = 3200


def _unpack_bf16x2(p):
    pu = lax.bitcast_convert_type(p, jnp.uint32)
    lo = lax.bitcast_convert_type(pu << 16, jnp.float32)
    hi = lax.bitcast_convert_type(pu & jnp.uint32(0xFFFF0000), jnp.float32)
    return jnp.concatenate([lo, hi], axis=1)


def _mlp_body_first(gp_ref, ea_ref, we_ref, w2_ref, b2_ref, o_ref):
    gp = gp_ref[...]
    h = (_unpack_bf16x2(gp[:, :LATENT // 2])
         + _unpack_bf16x2(gp[:, LATENT // 2:])
         + lax.dot_general(ea_ref[...], we_ref[...],
                           ((((0,), (0,)), ((), ()))),
                           preferred_element_type=jnp.float32))
    h = jnp.maximum(h, 0.0)
    o_ref[...] = jnp.maximum(
        jnp.dot(h, w2_ref[...], preferred_element_type=jnp.float32)
        + b2_ref[...], 0.0)


def _mlp_body_chained(prev_ref, gp_ref, ea_ref, we_ref, w2_ref,
                      b2_ref, o_ref):
    del prev_ref
    _mlp_body_first(gp_ref, ea_ref, we_ref, w2_ref, b2_ref, o_ref)


@functools.cache
def _build_mlp(row0, nblocks, first):
    gather_specs = [
        pl.BlockSpec((_BE, LATENT), lambda i: (i, 0)),
    ]
    shared_specs = [
        pl.BlockSpec((D_EDGE, _BE), lambda i, row0=row0: (0, row0 + i)),
        pl.BlockSpec((D_EDGE, LATENT), lambda i: (0, 0)),
        pl.BlockSpec((LATENT, LATENT), lambda i: (0, 0)),
        pl.BlockSpec((1, LATENT), lambda i: (0, 0)),
    ]
    out_spec = pl.BlockSpec((_BE, LATENT), lambda i, row0=row0: (row0 + i, 0))
    out_shape = jax.ShapeDtypeStruct((N_EDGES, LATENT), jnp.float32)
    if first:
        return pl.pallas_call(
            _mlp_body_first,
            grid=(nblocks,),
            in_specs=gather_specs + shared_specs,
            out_specs=out_spec,
            out_shape=out_shape,
        )
    return pl.pallas_call(
        _mlp_body_chained,
        grid=(nblocks,),
        in_specs=[pl.BlockSpec(memory_space=pl.ANY)] + gather_specs
        + shared_specs,
        out_specs=out_spec,
        out_shape=out_shape,
        input_output_aliases={0: 0},
    )


def kernel(x, edge_index, edge_attr, u, batch, W1, b1, W2, b2):
    src_all = edge_index[0].astype(jnp.int32)
    dst_all = edge_index[1].astype(jnp.int32)
    batch2d = batch.astype(jnp.int32).reshape(N_NODES, 1)
    wa = W1[0:D_FEAT]
    wb = W1[D_FEAT:2 * D_FEAT]
    we = W1[2 * D_FEAT:2 * D_FEAT + D_EDGE]
    wu = W1[2 * D_FEAT + D_EDGE:]
    a_tab, b_tab = _node_tables(x, batch2d, u, wa, wb, wu,
                                b1.reshape(1, LATENT))
    b2r = b2.reshape(1, LATENT)
    ea_t = edge_attr.T
    out = None
    lo = 0
    blk0 = 0
    for k in range(_K):
        n_real = _SLICE_REAL[k]
        nchunk = _SLICE_NCHUNK[k]
        n_pad_edges = _NW * nchunk * _CB
        pad = (jnp.arange(n_pad_edges - n_real, dtype=jnp.int32) % N_NODES)
        src = jnp.concatenate([src_all[lo:lo + n_real], pad]
                              ).reshape(_NW, nchunk, _CB)
        dst = jnp.concatenate([dst_all[lo:lo + n_real], pad]
                              ).reshape(_NW, nchunk, _CB)
        gp = _build_gather(nchunk)(a_tab, b_tab, src, dst)
        nblocks = n_real // _BE
        if k == 0:
            out = _build_mlp(blk0, nblocks, True)(gp, ea_t, we, W2, b2r)
        else:
            out = _build_mlp(blk0, nblocks, False)(out, gp, ea_t, we, W2, b2r)
        lo += n_real
        blk0 += nblocks
    return out
